# Initial kernel scaffold; baseline (speedup 1.0000x reference)
import jax
import jax.numpy as jnp
from jax.experimental import pallas as pl


def kernel(features, weight, bias, utilities, running_mean, ages):
    raise NotImplementedError("write your pallas kernel here")


# speedup vs baseline: 1.3308x; 1.3308x over previous
import jax
import jax.numpy as jnp
from jax.experimental import pallas as pl
from jax.experimental.pallas import tpu as pltpu

_DECAY = 0.9
_OMD = 1.0 - _DECAY
_MATURITY = 500
_REINIT_DIV = 100


def _feat_kernel(f_ref, rm_ref, rmnew_ref, cas_ref, *, inv_b):
    f = f_ref[...]
    colsum = jnp.sum(f, axis=0, keepdims=True)
    rm_new = _DECAY * rm_ref[...] + _OMD * (colsum * inv_b)
    rmnew_ref[...] = rm_new
    cas_ref[...] = jnp.sum(jnp.abs(f - rm_new), axis=0, keepdims=True)


def _wsum_kernel(w_ref, outg_ref, inc_ref):
    aw = jnp.abs(w_ref[...])
    outg_ref[...] = jnp.sum(aw, axis=0)[None, None, :]
    inc_ref[...] = jnp.sum(aw, axis=1, keepdims=True)


def _util_kernel(cas_ref, outg_ref, inc_ref, u_ref, unew_ref, *, inv_bin):
    outgoing = jnp.sum(outg_ref[...], axis=0)
    c = jnp.sum(outgoing * cas_ref[...]) * inv_bin
    inst = c / (inc_ref[...] + 1e-8)
    unew_ref[...] = _DECAY * u_ref[...] + _OMD * inst


def _select_kernel(ucol_ref, urow_ref, agesc_ref, agesr_ref, bias_ref,
                   mask_ref, bnew_ref, anew_ref, *, out_n, chunk):
    ages_row = agesr_ref[...]
    mature_row = ages_row > _MATURITY
    num_mature = jnp.sum(mature_row.astype(jnp.int32))
    num_reinit = num_mature // _REINIT_DIV

    u_row = urow_ref[...]
    j_idx = jax.lax.broadcasted_iota(jnp.int32, (chunk, out_n), 1)
    for cstart in range(0, out_n, chunk):
        u_chunk = ucol_ref[pl.ds(cstart, chunk), :]
        i_idx = cstart + jax.lax.broadcasted_iota(
            jnp.int32, (chunk, out_n), 0)
        before = (u_row < u_chunk) | ((u_row == u_chunk) & (j_idx < i_idx))
        before = before & mature_row
        cnt = jnp.sum(before.astype(jnp.int32), axis=1, keepdims=True)
        mature_chunk = agesc_ref[pl.ds(cstart, chunk), :] > _MATURITY
        sel = mature_chunk & (cnt < num_reinit)
        mask_ref[pl.ds(cstart, chunk), :] = sel.astype(jnp.int32)

    sel_all = mask_ref[...] != 0
    bnew_ref[...] = jnp.where(sel_all, 0.0, bias_ref[...])
    ages_col = agesc_ref[...]
    anew_ref[...] = jnp.where(sel_all, 0, ages_col) + 1


def _wzero_kernel(w_ref, mask_ref, out_ref):
    sel = mask_ref[...] != 0
    out_ref[...] = jnp.where(sel, 0.0, w_ref[...])


def kernel(features, weight, bias, utilities, running_mean, ages):
    B, IN = features.shape
    OUT = weight.shape[0]
    CB = 512
    RB = 512
    NB = OUT // RB
    f32 = jnp.float32

    rm2 = running_mean.reshape(1, IN)

    rm_new2, cas = pl.pallas_call(
        lambda f, r, o1, o2: _feat_kernel(f, r, o1, o2, inv_b=1.0 / B),
        grid=(IN // CB,),
        in_specs=[
            pl.BlockSpec((B, CB), lambda j: (0, j)),
            pl.BlockSpec((1, CB), lambda j: (0, j)),
        ],
        out_specs=[
            pl.BlockSpec((1, CB), lambda j: (0, j)),
            pl.BlockSpec((1, CB), lambda j: (0, j)),
        ],
        out_shape=[
            jax.ShapeDtypeStruct((1, IN), f32),
            jax.ShapeDtypeStruct((1, IN), f32),
        ],
        compiler_params=pltpu.CompilerParams(
            dimension_semantics=("parallel",)),
    )(features, rm2)

    outg_part, incoming = pl.pallas_call(
        _wsum_kernel,
        grid=(NB,),
        in_specs=[pl.BlockSpec((RB, IN), lambda i: (i, 0))],
        out_specs=[
            pl.BlockSpec((1, 1, IN), lambda i: (i, 0, 0)),
            pl.BlockSpec((RB, 1), lambda i: (i, 0)),
        ],
        out_shape=[
            jax.ShapeDtypeStruct((NB, 1, IN), f32),
            jax.ShapeDtypeStruct((OUT, 1), f32),
        ],
        compiler_params=pltpu.CompilerParams(
            dimension_semantics=("parallel",)),
    )(weight)

    u_col = pl.pallas_call(
        lambda c, o, i, u, un: _util_kernel(
            c, o, i, u, un, inv_bin=1.0 / (B * IN)),
        out_shape=jax.ShapeDtypeStruct((OUT, 1), f32),
    )(cas, outg_part, incoming, utilities.reshape(OUT, 1))

    u_row = u_col.reshape(1, OUT)
    ages_col = ages.reshape(OUT, 1)
    ages_row = ages.reshape(1, OUT)

    CH = 256
    mask_col, bias_new, ages_new = pl.pallas_call(
        lambda uc, ur, ac, ar, b, m, bn, an: _select_kernel(
            uc, ur, ac, ar, b, m, bn, an, out_n=OUT, chunk=CH),
        out_shape=[
            jax.ShapeDtypeStruct((OUT, 1), jnp.int32),
            jax.ShapeDtypeStruct((OUT, 1), f32),
            jax.ShapeDtypeStruct((OUT, 1), ages.dtype),
        ],
    )(u_col, u_row, ages_col, ages_row, bias.reshape(OUT, 1))

    weight_new = pl.pallas_call(
        _wzero_kernel,
        grid=(NB,),
        in_specs=[
            pl.BlockSpec((RB, IN), lambda i: (i, 0)),
            pl.BlockSpec((RB, 1), lambda i: (i, 0)),
        ],
        out_specs=pl.BlockSpec((RB, IN), lambda i: (i, 0)),
        out_shape=jax.ShapeDtypeStruct((OUT, IN), f32),
        compiler_params=pltpu.CompilerParams(
            dimension_semantics=("parallel",)),
    )(weight, mask_col)

    return (weight_new, bias_new.reshape(OUT), u_col.reshape(OUT),
            rm_new2.reshape(IN), ages_new.reshape(OUT))


# speedup vs baseline: 1.5668x; 1.1773x over previous
import jax
import jax.numpy as jnp
from jax.experimental import pallas as pl
from jax.experimental.pallas import tpu as pltpu

_DECAY = 0.9
_OMD = 1.0 - _DECAY
_MATURITY = 500
_REINIT_DIV = 100
_KMAX = 64


def _feat_kernel(f_ref, rm_ref, rmnew_ref, cas_ref, *, inv_b):
    f = f_ref[...]
    colsum = jnp.sum(f, axis=0, keepdims=True)
    rm_new = _DECAY * rm_ref[...] + _OMD * (colsum * inv_b)
    rmnew_ref[...] = rm_new
    cas_ref[...] = jnp.sum(jnp.abs(f - rm_new), axis=0, keepdims=True)


def _wsum_copy_kernel(w_ref, outg_ref, inc_ref, wcopy_ref):
    w = w_ref[...]
    wcopy_ref[...] = w
    aw = jnp.abs(w)
    outg_ref[...] = jnp.sum(aw, axis=0)[None, None, :]
    inc_ref[...] = jnp.sum(aw, axis=1, keepdims=True)


def _util_kernel(cas_ref, outg_ref, inc_ref, u_ref, unew_ref, *, inv_bin):
    outgoing = jnp.sum(outg_ref[...], axis=0)
    c = jnp.sum(outgoing * cas_ref[...]) * inv_bin
    inst = c / (inc_ref[...] + 1e-8)
    unew_ref[...] = _DECAY * u_ref[...] + _OMD * inst


def _select_kernel(ucol_ref, urow_ref, agesc_ref, agesr_ref, bias_ref,
                   idx_ref, bnew_ref, anew_ref, cnt_ref, sel_ref,
                   *, out_n, chunk):
    ages_row = agesr_ref[...]
    mature_row = ages_row > _MATURITY
    num_mature = jnp.sum(mature_row.astype(jnp.int32))
    num_reinit = num_mature // _REINIT_DIV

    u_row = urow_ref[...]
    j_idx = jax.lax.broadcasted_iota(jnp.int32, (chunk, out_n), 1)
    for cstart in range(0, out_n, chunk):
        u_chunk = ucol_ref[pl.ds(cstart, chunk), :]
        i_idx = cstart + jax.lax.broadcasted_iota(
            jnp.int32, (chunk, out_n), 0)
        before = (u_row < u_chunk) | ((u_row == u_chunk) & (j_idx < i_idx))
        before = before & mature_row
        cnt = jnp.sum(before.astype(jnp.int32), axis=1, keepdims=True)
        cnt_ref[pl.ds(cstart, chunk), :] = cnt
        mature_chunk = agesc_ref[pl.ds(cstart, chunk), :] > _MATURITY
        sel = mature_chunk & (cnt < num_reinit)
        sel_ref[pl.ds(cstart, chunk), :] = sel.astype(jnp.int32)

    sel_all = sel_ref[...] != 0
    bnew_ref[...] = jnp.where(sel_all, 0.0, bias_ref[...])
    anew_ref[...] = jnp.where(sel_all, 0, agesc_ref[...]) + 1

    k_row = jax.lax.broadcasted_iota(jnp.int32, (out_n, _KMAX), 1)
    i_col = jax.lax.broadcasted_iota(jnp.int32, (out_n, _KMAX), 0)
    hit = sel_all & (cnt_ref[...] == k_row)
    idx_ref[...] = jnp.sum(jnp.where(hit, i_col + 1, 0),
                           axis=0, keepdims=True) - 1


def _zero_rows_kernel(idx_ref, w_ref, out_ref, zeros_ref, sems):
    zeros_ref[...] = jnp.zeros_like(zeros_ref)
    for k in range(_KMAX):
        @pl.when(idx_ref[k] >= 0)
        def _():
            pltpu.make_async_copy(
                zeros_ref, out_ref.at[pl.ds(idx_ref[k], 1), :], sems.at[k],
            ).start()
    for k in range(_KMAX):
        @pl.when(idx_ref[k] >= 0)
        def _():
            pltpu.make_async_copy(
                zeros_ref, out_ref.at[pl.ds(idx_ref[k], 1), :], sems.at[k],
            ).wait()


def kernel(features, weight, bias, utilities, running_mean, ages):
    B, IN = features.shape
    OUT = weight.shape[0]
    CB = 512
    RB = 512
    NB = OUT // RB
    f32 = jnp.float32

    rm2 = running_mean.reshape(1, IN)

    rm_new2, cas = pl.pallas_call(
        lambda f, r, o1, o2: _feat_kernel(f, r, o1, o2, inv_b=1.0 / B),
        grid=(IN // CB,),
        in_specs=[
            pl.BlockSpec((B, CB), lambda j: (0, j)),
            pl.BlockSpec((1, CB), lambda j: (0, j)),
        ],
        out_specs=[
            pl.BlockSpec((1, CB), lambda j: (0, j)),
            pl.BlockSpec((1, CB), lambda j: (0, j)),
        ],
        out_shape=[
            jax.ShapeDtypeStruct((1, IN), f32),
            jax.ShapeDtypeStruct((1, IN), f32),
        ],
        compiler_params=pltpu.CompilerParams(
            dimension_semantics=("parallel",)),
    )(features, rm2)

    outg_part, incoming, w_copy = pl.pallas_call(
        _wsum_copy_kernel,
        grid=(NB,),
        in_specs=[pl.BlockSpec((RB, IN), lambda i: (i, 0))],
        out_specs=[
            pl.BlockSpec((1, 1, IN), lambda i: (i, 0, 0)),
            pl.BlockSpec((RB, 1), lambda i: (i, 0)),
            pl.BlockSpec((RB, IN), lambda i: (i, 0)),
        ],
        out_shape=[
            jax.ShapeDtypeStruct((NB, 1, IN), f32),
            jax.ShapeDtypeStruct((OUT, 1), f32),
            jax.ShapeDtypeStruct((OUT, IN), f32),
        ],
        compiler_params=pltpu.CompilerParams(
            dimension_semantics=("parallel",)),
    )(weight)

    u_col = pl.pallas_call(
        lambda c, o, i, u, un: _util_kernel(
            c, o, i, u, un, inv_bin=1.0 / (B * IN)),
        out_shape=jax.ShapeDtypeStruct((OUT, 1), f32),
    )(cas, outg_part, incoming, utilities.reshape(OUT, 1))

    u_row = u_col.reshape(1, OUT)
    ages_col = ages.reshape(OUT, 1)
    ages_row = ages.reshape(1, OUT)

    CH = 256
    idx_row, bias_new, ages_new = pl.pallas_call(
        lambda uc, ur, ac, ar, b, m, bn, an, cs, ss: _select_kernel(
            uc, ur, ac, ar, b, m, bn, an, cs, ss, out_n=OUT, chunk=CH),
        out_shape=[
            jax.ShapeDtypeStruct((1, _KMAX), jnp.int32),
            jax.ShapeDtypeStruct((OUT, 1), f32),
            jax.ShapeDtypeStruct((OUT, 1), ages.dtype),
        ],
        scratch_shapes=[
            pltpu.VMEM((OUT, 1), jnp.int32),
            pltpu.VMEM((OUT, 1), jnp.int32),
        ],
    )(u_col, u_row, ages_col, ages_row, bias.reshape(OUT, 1))

    weight_new = pl.pallas_call(
        _zero_rows_kernel,
        in_specs=[
            pl.BlockSpec(memory_space=pltpu.MemorySpace.SMEM),
            pl.BlockSpec(memory_space=pltpu.MemorySpace.HBM),
        ],
        out_specs=pl.BlockSpec(memory_space=pltpu.MemorySpace.HBM),
        out_shape=jax.ShapeDtypeStruct((OUT, IN), f32),
        scratch_shapes=[
            pltpu.VMEM((1, IN), f32),
            pltpu.SemaphoreType.DMA((_KMAX,)),
        ],
        input_output_aliases={1: 0},
    )(idx_row.reshape(_KMAX), w_copy)

    return (weight_new, bias_new.reshape(OUT), u_col.reshape(OUT),
            rm_new2.reshape(IN), ages_new.reshape(OUT))


# speedup vs baseline: 1.9354x; 1.2353x over previous
import jax
import jax.numpy as jnp
from jax.experimental import pallas as pl
from jax.experimental.pallas import tpu as pltpu

_DECAY = 0.9
_OMD = 1.0 - _DECAY
_MATURITY = 500
_REINIT_DIV = 100
_KMAX = 64
_I32MAX = 2**31 - 1


def _feat_kernel(f_ref, rm_ref, rmnew_ref, cas_ref, *, inv_b):
    f = f_ref[...]
    colsum = jnp.sum(f, axis=0, keepdims=True)
    rm_new = _DECAY * rm_ref[...] + _OMD * (colsum * inv_b)
    rmnew_ref[...] = rm_new
    cas_ref[...] = jnp.sum(jnp.abs(f - rm_new), axis=0, keepdims=True)


def _wsum_copy_kernel(w_ref, outg_ref, inc_ref, wcopy_ref):
    w = w_ref[...]
    wcopy_ref[...] = w
    aw = jnp.abs(w)
    outg_ref[...] = jnp.sum(aw, axis=0)[None, None, :]
    inc_ref[...] = jnp.sum(aw, axis=1, keepdims=True)


def _excl_prefix_sum_row(x):
    n = x.shape[1]
    s = x
    sh = 1
    while sh < n:
        shifted = jnp.concatenate(
            [jnp.zeros((1, sh), x.dtype), s[:, :n - sh]], axis=1)
        s = s + shifted
        sh *= 2
    return s - x


def _update_select_kernel(cas_ref, outg_ref, incr_ref, u_ref, ages_ref,
                          bias_ref, unew_ref, bnew_ref, anew_ref, idx_ref,
                          *, inv_bin, out_n):
    i32 = jnp.int32
    outgoing = jnp.sum(outg_ref[...], axis=0)
    c = jnp.sum(outgoing * cas_ref[...]) * inv_bin
    u_new = _DECAY * u_ref[...] + _OMD * (c / (incr_ref[...] + 1e-8))
    unew_ref[...] = u_new

    ages = ages_ref[...]
    mature = ages > _MATURITY
    num_mature = jnp.sum(mature.astype(i32))
    num_reinit = num_mature // _REINIT_DIV
    r = jnp.maximum(num_reinit, 1)

    key = jnp.where(mature, jax.lax.bitcast_convert_type(u_new, i32),
                    i32(_I32MAX))

    def bisect(_, lohi):
        lo, hi = lohi
        mid = (lo + hi) // 2
        ge = jnp.sum((key <= mid).astype(i32)) >= r
        return jnp.where(ge, lo, mid), jnp.where(ge, mid, hi)

    _, v = jax.lax.fori_loop(0, 31, bisect, (i32(-1), i32(_I32MAX)))

    c_lt = jnp.sum((key < v).astype(i32))
    tie = key == v
    tie_rank = _excl_prefix_sum_row(tie.astype(i32))
    sel = ((key < v) | (tie & (tie_rank < (r - c_lt)))) & (num_reinit > 0)

    bnew_ref[...] = jnp.where(sel, 0.0, bias_ref[...])
    anew_ref[...] = jnp.where(sel, 0, ages) + 1

    slot = _excl_prefix_sum_row(sel.astype(i32))
    k_col = jax.lax.broadcasted_iota(i32, (_KMAX, out_n), 0)
    i_row = jax.lax.broadcasted_iota(i32, (_KMAX, out_n), 1)
    hit = sel & (slot == k_col)
    idx_ref[...] = jnp.sum(jnp.where(hit, i_row + 1, 0),
                           axis=1, keepdims=True) - 1


def _zero_rows_kernel(idx_ref, w_ref, out_ref, zeros_ref, sems):
    zeros_ref[...] = jnp.zeros_like(zeros_ref)
    for k in range(_KMAX):
        @pl.when(idx_ref[k] >= 0)
        def _():
            pltpu.make_async_copy(
                zeros_ref, out_ref.at[pl.ds(idx_ref[k], 1), :], sems.at[k],
            ).start()
    for k in range(_KMAX):
        @pl.when(idx_ref[k] >= 0)
        def _():
            pltpu.make_async_copy(
                zeros_ref, out_ref.at[pl.ds(idx_ref[k], 1), :], sems.at[k],
            ).wait()


def kernel(features, weight, bias, utilities, running_mean, ages):
    B, IN = features.shape
    OUT = weight.shape[0]
    CB = 512
    RB = 512
    NB = OUT // RB
    f32 = jnp.float32

    rm2 = running_mean.reshape(1, IN)

    rm_new2, cas = pl.pallas_call(
        lambda f, r, o1, o2: _feat_kernel(f, r, o1, o2, inv_b=1.0 / B),
        grid=(IN // CB,),
        in_specs=[
            pl.BlockSpec((B, CB), lambda j: (0, j)),
            pl.BlockSpec((1, CB), lambda j: (0, j)),
        ],
        out_specs=[
            pl.BlockSpec((1, CB), lambda j: (0, j)),
            pl.BlockSpec((1, CB), lambda j: (0, j)),
        ],
        out_shape=[
            jax.ShapeDtypeStruct((1, IN), f32),
            jax.ShapeDtypeStruct((1, IN), f32),
        ],
        compiler_params=pltpu.CompilerParams(
            dimension_semantics=("parallel",)),
    )(features, rm2)

    outg_part, incoming, w_copy = pl.pallas_call(
        _wsum_copy_kernel,
        grid=(NB,),
        in_specs=[pl.BlockSpec((RB, IN), lambda i: (i, 0))],
        out_specs=[
            pl.BlockSpec((1, 1, IN), lambda i: (i, 0, 0)),
            pl.BlockSpec((RB, 1), lambda i: (i, 0)),
            pl.BlockSpec((RB, IN), lambda i: (i, 0)),
        ],
        out_shape=[
            jax.ShapeDtypeStruct((NB, 1, IN), f32),
            jax.ShapeDtypeStruct((OUT, 1), f32),
            jax.ShapeDtypeStruct((OUT, IN), f32),
        ],
        compiler_params=pltpu.CompilerParams(
            dimension_semantics=("parallel",)),
    )(weight)

    u_new, bias_new, ages_new, idx_col = pl.pallas_call(
        lambda ca, og, ic, u, ag, b, un, bn, an, ix: _update_select_kernel(
            ca, og, ic, u, ag, b, un, bn, an, ix,
            inv_bin=1.0 / (B * IN), out_n=OUT),
        out_shape=[
            jax.ShapeDtypeStruct((1, OUT), f32),
            jax.ShapeDtypeStruct((1, OUT), f32),
            jax.ShapeDtypeStruct((1, OUT), ages.dtype),
            jax.ShapeDtypeStruct((_KMAX, 1), jnp.int32),
        ],
    )(cas, outg_part, incoming.reshape(1, OUT), utilities.reshape(1, OUT),
      ages.reshape(1, OUT), bias.reshape(1, OUT))

    weight_new = pl.pallas_call(
        _zero_rows_kernel,
        in_specs=[
            pl.BlockSpec(memory_space=pltpu.MemorySpace.SMEM),
            pl.BlockSpec(memory_space=pltpu.MemorySpace.HBM),
        ],
        out_specs=pl.BlockSpec(memory_space=pltpu.MemorySpace.HBM),
        out_shape=jax.ShapeDtypeStruct((OUT, IN), f32),
        scratch_shapes=[
            pltpu.VMEM((1, IN), f32),
            pltpu.SemaphoreType.DMA((_KMAX,)),
        ],
        input_output_aliases={1: 0},
    )(idx_col.reshape(_KMAX), w_copy)

    return (weight_new, bias_new.reshape(OUT), u_new.reshape(OUT),
            rm_new2.reshape(IN), ages_new.reshape(OUT))


# speedup vs baseline: 1.9692x; 1.0175x over previous
import jax
import jax.numpy as jnp
from jax.experimental import pallas as pl
from jax.experimental.pallas import tpu as pltpu

_DECAY = 0.9
_OMD = 1.0 - _DECAY
_MATURITY = 500
_REINIT_DIV = 100
_KMAX = 64
_I32MAX = 2**31 - 1


def _feat_kernel(f_ref, rm_ref, rmnew_ref, cas_ref, *, inv_b):
    f = f_ref[...]
    colsum = jnp.sum(f, axis=0, keepdims=True)
    rm_new = _DECAY * rm_ref[...] + _OMD * (colsum * inv_b)
    rmnew_ref[...] = rm_new
    cas_ref[...] = jnp.sum(jnp.abs(f - rm_new), axis=0, keepdims=True)


def _wsum_copy_kernel(w_ref, outg_ref, inc_ref, wcopy_ref):
    w = w_ref[...]
    wcopy_ref[...] = w
    aw = jnp.abs(w)
    outg_ref[...] = jnp.sum(aw, axis=0)[None, None, :]
    inc_ref[...] = jnp.sum(aw, axis=1, keepdims=True)


def _excl_prefix_sum_row(x):
    n = x.shape[1]
    s = x
    sh = 1
    while sh < n:
        shifted = jnp.concatenate(
            [jnp.zeros((1, sh), x.dtype), s[:, :n - sh]], axis=1)
        s = s + shifted
        sh *= 2
    return s - x


def _update_select_kernel(cas_ref, outg_ref, incr_ref, u_ref, ages_ref,
                          bias_ref, w_ref, unew_ref, bnew_ref, anew_ref,
                          wout_ref, zeros_ref, sems, *, inv_bin, out_n):
    i32 = jnp.int32
    outgoing = jnp.sum(outg_ref[...], axis=0)
    c = jnp.sum(outgoing * cas_ref[...]) * inv_bin
    u_new = _DECAY * u_ref[...] + _OMD * (c / (incr_ref[...] + 1e-8))
    unew_ref[...] = u_new

    ages = ages_ref[...]
    mature = ages > _MATURITY
    num_mature = jnp.sum(mature.astype(i32))
    num_reinit = num_mature // _REINIT_DIV
    r = jnp.maximum(num_reinit, 1)

    key = jnp.where(mature, jax.lax.bitcast_convert_type(u_new, i32),
                    i32(_I32MAX))

    def bisect(_, lohi):
        lo, hi = lohi
        mid = (lo + hi) // 2
        ge = jnp.sum((key <= mid).astype(i32)) >= r
        return jnp.where(ge, lo, mid), jnp.where(ge, mid, hi)

    _, v = jax.lax.fori_loop(0, 31, bisect, (i32(-1), i32(_I32MAX)))

    c_lt = jnp.sum((key < v).astype(i32))
    tie = key == v
    tie_rank = _excl_prefix_sum_row(tie.astype(i32))
    sel = ((key < v) | (tie & (tie_rank < (r - c_lt)))) & (num_reinit > 0)

    bnew_ref[...] = jnp.where(sel, 0.0, bias_ref[...])
    anew_ref[...] = jnp.where(sel, 0, ages) + 1

    slot = _excl_prefix_sum_row(sel.astype(i32))
    i_row = jax.lax.broadcasted_iota(i32, (1, out_n), 1)
    zeros_ref[...] = jnp.zeros_like(zeros_ref)
    idxs = []
    for k in range(_KMAX):
        hit = sel & (slot == k)
        idx_k = jnp.sum(jnp.where(hit, i_row + 1, 0)) - 1
        idxs.append(idx_k)

        @pl.when(idx_k >= 0)
        def _():
            pltpu.make_async_copy(
                zeros_ref, wout_ref.at[pl.ds(idx_k, 1), :], sems.at[k],
            ).start()
    for k in range(_KMAX):
        @pl.when(idxs[k] >= 0)
        def _():
            pltpu.make_async_copy(
                zeros_ref, wout_ref.at[pl.ds(idxs[k], 1), :], sems.at[k],
            ).wait()


def kernel(features, weight, bias, utilities, running_mean, ages):
    B, IN = features.shape
    OUT = weight.shape[0]
    CB = 512
    RB = 512
    NB = OUT // RB
    f32 = jnp.float32

    rm2 = running_mean.reshape(1, IN)

    rm_new2, cas = pl.pallas_call(
        lambda f, r, o1, o2: _feat_kernel(f, r, o1, o2, inv_b=1.0 / B),
        grid=(IN // CB,),
        in_specs=[
            pl.BlockSpec((B, CB), lambda j: (0, j)),
            pl.BlockSpec((1, CB), lambda j: (0, j)),
        ],
        out_specs=[
            pl.BlockSpec((1, CB), lambda j: (0, j)),
            pl.BlockSpec((1, CB), lambda j: (0, j)),
        ],
        out_shape=[
            jax.ShapeDtypeStruct((1, IN), f32),
            jax.ShapeDtypeStruct((1, IN), f32),
        ],
        compiler_params=pltpu.CompilerParams(
            dimension_semantics=("parallel",)),
    )(features, rm2)

    outg_part, incoming, w_copy = pl.pallas_call(
        _wsum_copy_kernel,
        grid=(NB,),
        in_specs=[pl.BlockSpec((RB, IN), lambda i: (i, 0))],
        out_specs=[
            pl.BlockSpec((1, 1, IN), lambda i: (i, 0, 0)),
            pl.BlockSpec((RB, 1), lambda i: (i, 0)),
            pl.BlockSpec((RB, IN), lambda i: (i, 0)),
        ],
        out_shape=[
            jax.ShapeDtypeStruct((NB, 1, IN), f32),
            jax.ShapeDtypeStruct((OUT, 1), f32),
            jax.ShapeDtypeStruct((OUT, IN), f32),
        ],
        compiler_params=pltpu.CompilerParams(
            dimension_semantics=("parallel",)),
    )(weight)

    vm = pl.BlockSpec(memory_space=pltpu.MemorySpace.VMEM)
    u_new, bias_new, ages_new, weight_new = pl.pallas_call(
        lambda ca, og, ic, u, ag, b, w, un, bn, an, wo, zr, sm:
            _update_select_kernel(
                ca, og, ic, u, ag, b, w, un, bn, an, wo, zr, sm,
                inv_bin=1.0 / (B * IN), out_n=OUT),
        in_specs=[vm, vm, vm, vm, vm, vm,
                  pl.BlockSpec(memory_space=pltpu.MemorySpace.HBM)],
        out_specs=[vm, vm, vm,
                   pl.BlockSpec(memory_space=pltpu.MemorySpace.HBM)],
        out_shape=[
            jax.ShapeDtypeStruct((1, OUT), f32),
            jax.ShapeDtypeStruct((1, OUT), f32),
            jax.ShapeDtypeStruct((1, OUT), ages.dtype),
            jax.ShapeDtypeStruct((OUT, IN), f32),
        ],
        scratch_shapes=[
            pltpu.VMEM((1, IN), f32),
            pltpu.SemaphoreType.DMA((_KMAX,)),
        ],
        input_output_aliases={6: 3},
    )(cas, outg_part, incoming.reshape(1, OUT), utilities.reshape(1, OUT),
      ages.reshape(1, OUT), bias.reshape(1, OUT), w_copy)

    return (weight_new, bias_new.reshape(OUT), u_new.reshape(OUT),
            rm_new2.reshape(IN), ages_new.reshape(OUT))
